# baseline (speedup 1.0000x reference)
import functools

import jax
import jax.numpy as jnp
from jax import lax
from jax.experimental import pallas as pl
from jax.experimental.pallas import tpu as pltpu
from jax.experimental.pallas import tpu_sc as plsc

NC = 2
NS = 16
CHUNK = 125

_SELU_ALPHA = 1.6732632423543772
_SELU_SCALE = 1.0507009873554805


def _selu(v):
    return _SELU_SCALE * jnp.where(v > 0, v, _SELU_ALPHA * (jnp.exp(v) - 1.0))


def _matmul_tc(x, w, n_out):
    d = x.shape[1]
    h = w.shape[1]
    blk = 128
    grid = n_out // blk
    last = (x.shape[0] - 1) // blk

    def body(x_ref, w_ref, o_ref):
        o_ref[...] = jnp.dot(x_ref[...], w_ref[...],
                             preferred_element_type=jnp.float32)

    return pl.pallas_call(
        body,
        grid=(grid,),
        in_specs=[
            pl.BlockSpec((blk, d), lambda i: (jnp.minimum(i, last), 0)),
            pl.BlockSpec((d, h), lambda i: (0, 0)),
        ],
        out_specs=pl.BlockSpec((blk, h), lambda i: (i, 0)),
        out_shape=jax.ShapeDtypeStruct((n_out, h), jnp.float32),
    )(x, w)


def _message_passing_sc(rows, src_r, dst_r, b1rows, zeros):
    n, h = rows.shape
    hh = h // NC
    nchunk = src_r.shape[1]
    rps = n // NS
    mesh = plsc.VectorSubcoreMesh(core_axis_name="c", subcore_axis_name="s")

    @functools.partial(
        pl.kernel,
        out_type=jax.ShapeDtypeStruct((n, h), jnp.float32),
        mesh=mesh,
        scratch_types=[
            pltpu.VMEM((nchunk, CHUNK), jnp.int32),
            pltpu.VMEM((nchunk, CHUNK), jnp.int32),
            pltpu.VMEM((CHUNK, hh), jnp.float32),
            pltpu.VMEM((CHUNK, hh), jnp.float32),
            pltpu.VMEM_SHARED((n, hh), jnp.float32),
            pltpu.VMEM_SHARED((n, hh), jnp.float32),
            pltpu.VMEM_SHARED((n, hh), jnp.float32),
            pltpu.SemaphoreType.DMA,
            pltpu.SemaphoreType.DMA,
            pltpu.SemaphoreType.DMA,
            pltpu.SemaphoreType.DMA,
            pltpu.SemaphoreType.DMA,
        ],
        compiler_params=pltpu.CompilerParams(use_tc_tiling_on_sc=False),
    )
    def k(rows_hbm, src_hbm, dst_hbm, b1_hbm, zeros_hbm, out_hbm,
          src_v, dst_v, g0, g1, table, acc1, acc2,
          sg0, sg1, sg2, sg3, szero):
        g = (g0, g1)
        sg = (sg0, sg1)
        c = lax.axis_index("c")
        s = lax.axis_index("s")
        row0 = s * rps
        col0 = c * hh
        rows_sl = pl.ds(row0, rps)
        cols_sl = pl.ds(col0, hh)
        pltpu.async_copy(rows_hbm.at[rows_sl, cols_sl], table.at[rows_sl], sg0)
        pltpu.async_copy(b1_hbm.at[rows_sl, cols_sl], acc1.at[rows_sl], sg1)
        pltpu.async_copy(zeros_hbm.at[rows_sl, cols_sl], acc2.at[rows_sl],
                         szero)
        pltpu.async_copy(src_hbm.at[s], src_v, sg2)
        pltpu.async_copy(dst_hbm.at[s], dst_v, sg3)
        pltpu.make_async_copy(rows_hbm.at[rows_sl, cols_sl],
                              table.at[rows_sl], sg0).wait()
        pltpu.make_async_copy(b1_hbm.at[rows_sl, cols_sl],
                              acc1.at[rows_sl], sg1).wait()
        pltpu.make_async_copy(src_hbm.at[s], src_v, sg2).wait()
        pltpu.make_async_copy(dst_hbm.at[s], dst_v, sg3).wait()
        plsc.subcore_barrier()

        def round_(tab, acc):
            pltpu.make_async_copy(tab.at[src_v.at[0]], g[0], sg[0]).start()

            @pl.loop(0, nchunk // 2 - 1)
            def _(i):
                j = 2 * i
                cp1 = pltpu.make_async_copy(tab.at[src_v.at[j + 1]],
                                            g[1], sg[1])
                cp1.start()
                pltpu.make_async_copy(tab.at[src_v.at[j]], g[0], sg[0]).wait()
                pltpu.sync_copy(g[0], acc.at[dst_v.at[j]], add=True)
                cp0 = pltpu.make_async_copy(tab.at[src_v.at[j + 2]],
                                            g[0], sg[0])
                cp0.start()
                cp1.wait()
                pltpu.sync_copy(g[1], acc.at[dst_v.at[j + 1]], add=True)

            j = nchunk - 2
            cp1 = pltpu.make_async_copy(tab.at[src_v.at[j + 1]], g[1], sg[1])
            cp1.start()
            pltpu.make_async_copy(tab.at[src_v.at[j]], g[0], sg[0]).wait()
            pltpu.sync_copy(g[0], acc.at[dst_v.at[j]], add=True)
            cp1.wait()
            pltpu.sync_copy(g[1], acc.at[dst_v.at[j + 1]], add=True)

        round_(table, acc1)
        pltpu.make_async_copy(zeros_hbm.at[rows_sl, cols_sl],
                              acc2.at[rows_sl], szero).wait()
        plsc.subcore_barrier()
        round_(acc1, acc2)
        plsc.subcore_barrier()
        pltpu.sync_copy(acc2.at[rows_sl], out_hbm.at[rows_sl, cols_sl])

    return k(rows, src_r, dst_r, b1rows, zeros)


def _head_tc(agg2, n_real, W2, b2r, wg_row, bgr, Wf1, bf1r, Wf2, bf2r):
    np_ = agg2.shape[0]

    def body(a_ref, w2_ref, b2_ref, wg_ref, bg_ref,
             wf1_ref, bf1_ref, wf2_ref, bf2_ref, o_ref):
        h2 = jnp.dot(a_ref[...], w2_ref[...],
                     preferred_element_type=jnp.float32) + b2_ref[...]
        h2 = _selu(h2)
        gate = jnp.sum(h2 * wg_ref[...], axis=1, keepdims=True) + bg_ref[0, 0]
        row_ids = lax.broadcasted_iota(jnp.int32, (np_, 1), 0)
        gate = jnp.where(row_ids < n_real, gate, -jnp.inf)
        m = jnp.max(gate)
        e = jnp.exp(gate - m)
        denom = jnp.sum(e)
        readout = jnp.sum(e * h2, axis=0, keepdims=True) / denom
        z = _selu(jnp.dot(readout, wf1_ref[...],
                          preferred_element_type=jnp.float32) + bf1_ref[...])
        logit = jnp.dot(z, wf2_ref[...],
                        preferred_element_type=jnp.float32) + bf2_ref[...]
        o_ref[...] = jax.nn.sigmoid(logit)

    return pl.pallas_call(
        body,
        out_shape=jax.ShapeDtypeStruct((1, 1), jnp.float32),
    )(agg2, W2, b2r, wg_row, bgr, Wf1, bf1r, Wf2, bf2r)


def kernel(x, edge_index, W1, b1, W2, b2, Wg, bg, Wf1, bf1, Wf2, bf2):
    n = x.shape[0]
    e = edge_index.shape[1]
    h = W1.shape[1]
    eps = e // NS
    nchunk = eps // CHUNK
    np_ = ((n + NS * 8 - 1) // (NS * 8)) * (NS * 8)

    src_r = edge_index[0].reshape(NS, nchunk, CHUNK)
    dst_r = edge_index[1].reshape(NS, nchunk, CHUNK)
    b1rows = jnp.broadcast_to(b1.reshape(1, h), (np_, h))
    zeros = jnp.zeros((np_, h), jnp.float32)

    h_pre = _matmul_tc(x, W1, np_)
    agg2 = _message_passing_sc(h_pre, src_r, dst_r, b1rows, zeros)
    out = _head_tc(agg2, n, W2, b2.reshape(1, h), Wg.reshape(1, h),
                   bg.reshape(1, 1), Wf1, bf1.reshape(1, Wf1.shape[1]),
                   Wf2, bf2.reshape(1, 1))
    return out


# speedup vs baseline: 1.1637x; 1.1637x over previous
import functools

import jax
import jax.numpy as jnp
from jax import lax
from jax.experimental import pallas as pl
from jax.experimental.pallas import tpu as pltpu
from jax.experimental.pallas import tpu_sc as plsc

NC = 2
NS = 16
CHUNK = 125

_SELU_ALPHA = 1.6732632423543772
_SELU_SCALE = 1.0507009873554805


def _selu(v):
    return _SELU_SCALE * jnp.where(v > 0, v, _SELU_ALPHA * (jnp.exp(v) - 1.0))


def _matmul_tc(x, w):
    n, _ = x.shape
    h = w.shape[1]

    def body(x_ref, w_ref, o_ref):
        o_ref[...] = jnp.dot(x_ref[...], w_ref[...],
                             preferred_element_type=jnp.float32)

    return pl.pallas_call(
        body,
        out_shape=jax.ShapeDtypeStruct((n, h), jnp.float32),
    )(x, w)


def _message_passing_sc(rows, src_r, dst_r, b1rows, zeros):
    n, h = rows.shape
    hh = h // NC
    nchunk = src_r.shape[1]
    rps = n // NS
    mesh = plsc.VectorSubcoreMesh(core_axis_name="c", subcore_axis_name="s")

    @functools.partial(
        pl.kernel,
        out_type=jax.ShapeDtypeStruct((n, h), jnp.float32),
        mesh=mesh,
        scratch_types=[
            pltpu.VMEM((nchunk, CHUNK), jnp.int32),
            pltpu.VMEM((nchunk, CHUNK), jnp.int32),
            pltpu.VMEM((CHUNK, hh), jnp.float32),
            pltpu.VMEM((CHUNK, hh), jnp.float32),
            pltpu.VMEM_SHARED((n, hh), jnp.float32),
            pltpu.VMEM_SHARED((n, hh), jnp.float32),
            pltpu.VMEM_SHARED((n, hh), jnp.float32),
            pltpu.SemaphoreType.DMA,
            pltpu.SemaphoreType.DMA,
            pltpu.SemaphoreType.DMA,
            pltpu.SemaphoreType.DMA,
            pltpu.SemaphoreType.DMA,
        ],
        compiler_params=pltpu.CompilerParams(use_tc_tiling_on_sc=False),
    )
    def k(rows_hbm, src_hbm, dst_hbm, b1_hbm, zeros_hbm, out_hbm,
          src_v, dst_v, g0, g1, table, acc1, acc2,
          sg0, sg1, sg2, sg3, szero):
        g = (g0, g1)
        sg = (sg0, sg1)
        c = lax.axis_index("c")
        s = lax.axis_index("s")
        row0 = s * rps
        col0 = c * hh
        rows_sl = pl.ds(row0, rps)
        cols_sl = pl.ds(col0, hh)
        pltpu.async_copy(rows_hbm.at[rows_sl, cols_sl], table.at[rows_sl], sg0)
        pltpu.async_copy(b1_hbm.at[rows_sl, cols_sl], acc1.at[rows_sl], sg1)
        pltpu.async_copy(zeros_hbm.at[rows_sl, cols_sl], acc2.at[rows_sl],
                         szero)
        pltpu.async_copy(src_hbm.at[s], src_v, sg2)
        pltpu.async_copy(dst_hbm.at[s], dst_v, sg3)
        pltpu.make_async_copy(rows_hbm.at[rows_sl, cols_sl],
                              table.at[rows_sl], sg0).wait()
        pltpu.make_async_copy(b1_hbm.at[rows_sl, cols_sl],
                              acc1.at[rows_sl], sg1).wait()
        pltpu.make_async_copy(src_hbm.at[s], src_v, sg2).wait()
        pltpu.make_async_copy(dst_hbm.at[s], dst_v, sg3).wait()
        plsc.subcore_barrier()

        def round_(tab, acc):
            pltpu.make_async_copy(tab.at[src_v.at[0]], g[0], sg[0]).start()

            @pl.loop(0, nchunk // 2 - 1)
            def _(i):
                j = 2 * i
                cp1 = pltpu.make_async_copy(tab.at[src_v.at[j + 1]],
                                            g[1], sg[1])
                cp1.start()
                pltpu.make_async_copy(tab.at[src_v.at[j]], g[0], sg[0]).wait()
                pltpu.sync_copy(g[0], acc.at[dst_v.at[j]], add=True)
                cp0 = pltpu.make_async_copy(tab.at[src_v.at[j + 2]],
                                            g[0], sg[0])
                cp0.start()
                cp1.wait()
                pltpu.sync_copy(g[1], acc.at[dst_v.at[j + 1]], add=True)

            j = nchunk - 2
            cp1 = pltpu.make_async_copy(tab.at[src_v.at[j + 1]], g[1], sg[1])
            cp1.start()
            pltpu.make_async_copy(tab.at[src_v.at[j]], g[0], sg[0]).wait()
            pltpu.sync_copy(g[0], acc.at[dst_v.at[j]], add=True)
            cp1.wait()
            pltpu.sync_copy(g[1], acc.at[dst_v.at[j + 1]], add=True)

        round_(table, acc1)
        pltpu.make_async_copy(zeros_hbm.at[rows_sl, cols_sl],
                              acc2.at[rows_sl], szero).wait()
        plsc.subcore_barrier()
        round_(acc1, acc2)
        plsc.subcore_barrier()
        pltpu.sync_copy(acc2.at[rows_sl], out_hbm.at[rows_sl, cols_sl])

    return k(rows, src_r, dst_r, b1rows, zeros)


def _head_tc(agg2, n_real, W2, b2r, wg_row, bgr, Wf1, bf1r, Wf2, bf2r):
    np_ = agg2.shape[0]

    def body(a_ref, w2_ref, b2_ref, wg_ref, bg_ref,
             wf1_ref, bf1_ref, wf2_ref, bf2_ref, o_ref):
        h2 = jnp.dot(a_ref[...], w2_ref[...],
                     preferred_element_type=jnp.float32) + b2_ref[...]
        h2 = _selu(h2)
        gate = jnp.sum(h2 * wg_ref[...], axis=1, keepdims=True) + bg_ref[0, 0]
        row_ids = lax.broadcasted_iota(jnp.int32, (np_, 1), 0)
        gate = jnp.where(row_ids < n_real, gate, -jnp.inf)
        m = jnp.max(gate)
        e = jnp.exp(gate - m)
        denom = jnp.sum(e)
        readout = jnp.sum(e * h2, axis=0, keepdims=True) / denom
        z = _selu(jnp.dot(readout, wf1_ref[...],
                          preferred_element_type=jnp.float32) + bf1_ref[...])
        logit = jnp.dot(z, wf2_ref[...],
                        preferred_element_type=jnp.float32) + bf2_ref[...]
        o_ref[...] = jax.nn.sigmoid(logit)

    return pl.pallas_call(
        body,
        out_shape=jax.ShapeDtypeStruct((1, 1), jnp.float32),
    )(agg2, W2, b2r, wg_row, bgr, Wf1, bf1r, Wf2, bf2r)


def kernel(x, edge_index, W1, b1, W2, b2, Wg, bg, Wf1, bf1, Wf2, bf2):
    n = x.shape[0]
    e = edge_index.shape[1]
    h = W1.shape[1]
    eps = e // NS
    nchunk = eps // CHUNK
    np_ = ((n + NS * 8 - 1) // (NS * 8)) * (NS * 8)

    src_r = edge_index[0].reshape(NS, nchunk, CHUNK)
    dst_r = edge_index[1].reshape(NS, nchunk, CHUNK)
    b1rows = jnp.broadcast_to(b1.reshape(1, h), (np_, h))
    zeros = jnp.zeros((np_, h), jnp.float32)

    x_p = jnp.pad(x, ((0, np_ - n), (0, 0)))
    h_pre = _matmul_tc(x_p, W1)
    agg2 = _message_passing_sc(h_pre, src_r, dst_r, b1rows, zeros)
    out = _head_tc(agg2, n, W2, b2.reshape(1, h), Wg.reshape(1, h),
                   bg.reshape(1, 1), Wf1, bf1.reshape(1, Wf1.shape[1]),
                   Wf2, bf2.reshape(1, 1))
    return out


# speedup vs baseline: 1.1920x; 1.0244x over previous
import functools

import jax
import jax.numpy as jnp
from jax import lax
from jax.experimental import pallas as pl
from jax.experimental.pallas import tpu as pltpu
from jax.experimental.pallas import tpu_sc as plsc

NC = 2
NS = 16
CHUNK = 125

_SELU_ALPHA = 1.6732632423543772
_SELU_SCALE = 1.0507009873554805


def _selu(v):
    return _SELU_SCALE * jnp.where(v > 0, v, _SELU_ALPHA * (jnp.exp(v) - 1.0))


def _matmul_tc(x, w, n_out):
    n = x.shape[0]
    h = w.shape[1]

    def body(x_ref, w_ref, o_ref):
        o_ref[pl.ds(0, n), :] = jnp.dot(x_ref[...], w_ref[...],
                                        preferred_element_type=jnp.float32)

    return pl.pallas_call(
        body,
        out_shape=jax.ShapeDtypeStruct((n_out, h), jnp.float32),
    )(x, w)


def _message_passing_sc(rows, src_r, dst_r, b1rows, zeros):
    n, h = rows.shape
    hh = h // NC
    nchunk = src_r.shape[1]
    rps = n // NS
    mesh = plsc.VectorSubcoreMesh(core_axis_name="c", subcore_axis_name="s")

    @functools.partial(
        pl.kernel,
        out_type=jax.ShapeDtypeStruct((n, h), jnp.float32),
        mesh=mesh,
        scratch_types=[
            pltpu.VMEM((nchunk, CHUNK), jnp.int32),
            pltpu.VMEM((nchunk, CHUNK), jnp.int32),
            pltpu.VMEM((CHUNK, hh), jnp.float32),
            pltpu.VMEM((CHUNK, hh), jnp.float32),
            pltpu.VMEM_SHARED((n, hh), jnp.float32),
            pltpu.VMEM_SHARED((n, hh), jnp.float32),
            pltpu.VMEM_SHARED((n, hh), jnp.float32),
            pltpu.SemaphoreType.DMA,
            pltpu.SemaphoreType.DMA,
            pltpu.SemaphoreType.DMA,
            pltpu.SemaphoreType.DMA,
            pltpu.SemaphoreType.DMA,
        ],
        compiler_params=pltpu.CompilerParams(use_tc_tiling_on_sc=False),
    )
    def k(rows_hbm, src_hbm, dst_hbm, b1_hbm, zeros_hbm, out_hbm,
          src_v, dst_v, g0, g1, table, acc1, acc2,
          sg0, sg1, sg2, sg3, szero):
        g = (g0, g1)
        sg = (sg0, sg1)
        c = lax.axis_index("c")
        s = lax.axis_index("s")
        row0 = s * rps
        col0 = c * hh
        rows_sl = pl.ds(row0, rps)
        cols_sl = pl.ds(col0, hh)
        pltpu.async_copy(rows_hbm.at[rows_sl, cols_sl], table.at[rows_sl], sg0)
        pltpu.async_copy(b1_hbm.at[rows_sl, cols_sl], acc1.at[rows_sl], sg1)
        pltpu.async_copy(zeros_hbm.at[rows_sl, cols_sl], acc2.at[rows_sl],
                         szero)
        pltpu.async_copy(src_hbm.at[s], src_v, sg2)
        pltpu.async_copy(dst_hbm.at[s], dst_v, sg3)
        pltpu.make_async_copy(rows_hbm.at[rows_sl, cols_sl],
                              table.at[rows_sl], sg0).wait()
        pltpu.make_async_copy(b1_hbm.at[rows_sl, cols_sl],
                              acc1.at[rows_sl], sg1).wait()
        pltpu.make_async_copy(src_hbm.at[s], src_v, sg2).wait()
        pltpu.make_async_copy(dst_hbm.at[s], dst_v, sg3).wait()
        plsc.subcore_barrier()

        def round_(tab, acc):
            pltpu.make_async_copy(tab.at[src_v.at[0]], g[0], sg[0]).start()

            @pl.loop(0, nchunk // 2 - 1)
            def _(i):
                j = 2 * i
                cp1 = pltpu.make_async_copy(tab.at[src_v.at[j + 1]],
                                            g[1], sg[1])
                cp1.start()
                pltpu.make_async_copy(tab.at[src_v.at[j]], g[0], sg[0]).wait()
                pltpu.sync_copy(g[0], acc.at[dst_v.at[j]], add=True)
                cp0 = pltpu.make_async_copy(tab.at[src_v.at[j + 2]],
                                            g[0], sg[0])
                cp0.start()
                cp1.wait()
                pltpu.sync_copy(g[1], acc.at[dst_v.at[j + 1]], add=True)

            j = nchunk - 2
            cp1 = pltpu.make_async_copy(tab.at[src_v.at[j + 1]], g[1], sg[1])
            cp1.start()
            pltpu.make_async_copy(tab.at[src_v.at[j]], g[0], sg[0]).wait()
            pltpu.sync_copy(g[0], acc.at[dst_v.at[j]], add=True)
            cp1.wait()
            pltpu.sync_copy(g[1], acc.at[dst_v.at[j + 1]], add=True)

        round_(table, acc1)
        pltpu.make_async_copy(zeros_hbm.at[rows_sl, cols_sl],
                              acc2.at[rows_sl], szero).wait()
        plsc.subcore_barrier()
        round_(acc1, acc2)
        plsc.subcore_barrier()
        pltpu.sync_copy(acc2.at[rows_sl], out_hbm.at[rows_sl, cols_sl])

    return k(rows, src_r, dst_r, b1rows, zeros)


def _head_tc(agg2, n_real, W2, b2r, wg_row, bgr, Wf1, bf1r, Wf2, bf2r):
    np_ = agg2.shape[0]

    def body(a_ref, w2_ref, b2_ref, wg_ref, bg_ref,
             wf1_ref, bf1_ref, wf2_ref, bf2_ref, o_ref):
        h2 = jnp.dot(a_ref[...], w2_ref[...],
                     preferred_element_type=jnp.float32) + b2_ref[...]
        h2 = _selu(h2)
        gate = jnp.sum(h2 * wg_ref[...], axis=1, keepdims=True) + bg_ref[0, 0]
        row_ids = lax.broadcasted_iota(jnp.int32, (np_, 1), 0)
        gate = jnp.where(row_ids < n_real, gate, -jnp.inf)
        m = jnp.max(gate)
        e = jnp.exp(gate - m)
        denom = jnp.sum(e)
        readout = jnp.sum(e * h2, axis=0, keepdims=True) / denom
        z = _selu(jnp.dot(readout, wf1_ref[...],
                          preferred_element_type=jnp.float32) + bf1_ref[...])
        logit = jnp.dot(z, wf2_ref[...],
                        preferred_element_type=jnp.float32) + bf2_ref[...]
        o_ref[...] = jax.nn.sigmoid(logit)

    return pl.pallas_call(
        body,
        out_shape=jax.ShapeDtypeStruct((1, 1), jnp.float32),
    )(agg2, W2, b2r, wg_row, bgr, Wf1, bf1r, Wf2, bf2r)


def kernel(x, edge_index, W1, b1, W2, b2, Wg, bg, Wf1, bf1, Wf2, bf2):
    n = x.shape[0]
    e = edge_index.shape[1]
    h = W1.shape[1]
    eps = e // NS
    nchunk = eps // CHUNK
    np_ = ((n + NS * 8 - 1) // (NS * 8)) * (NS * 8)

    src_r = edge_index[0].reshape(NS, nchunk, CHUNK)
    dst_r = edge_index[1].reshape(NS, nchunk, CHUNK)
    b1rows = jnp.broadcast_to(b1.reshape(1, h), (np_, h))
    zeros = jnp.zeros((np_, h), jnp.float32)

    h_pre = _matmul_tc(x, W1, np_)
    agg2 = _message_passing_sc(h_pre, src_r, dst_r, b1rows, zeros)
    out = _head_tc(agg2, n, W2, b2.reshape(1, h), Wg.reshape(1, h),
                   bg.reshape(1, 1), Wf1, bf1.reshape(1, Wf1.shape[1]),
                   Wf2, bf2.reshape(1, 1))
    return out
